# Initial kernel scaffold; baseline (speedup 1.0000x reference)
import jax
import jax.numpy as jnp
from jax.experimental import pallas as pl


def kernel(x, tables):
    raise NotImplementedError("write your pallas kernel here")


# speedup vs baseline: 14.5732x; 14.5732x over previous
import functools

import numpy as np
import jax
import jax.numpy as jnp
from jax import lax
from jax.experimental import pallas as pl
from jax.experimental.pallas import tpu as pltpu
from jax.experimental.pallas import tpu_sc as plsc

_F = 26
_V = 4000
_SUM = _F * _V
_D = 16
_B = 4096
_NPAIR = (_F * (_F - 1)) // 2
_NW = 32
_BPW = _B // _NW
_G = 5
_NGROUPS = _NPAIR // _G

_IU, _JU = np.triu_indices(_F, k=1)
_CONST_A = (_JU * _SUM + _V * _IU).astype(np.int32)
_CONST_B = (_IU * _SUM + _V * _JU).astype(np.int32)


def _ffm_body(tables_hbm, idx_hbm, out_hbm, idx_v, rows_v, out_v, gsem):
    wid = lax.axis_index("s") * 2 + lax.axis_index("c")
    b0 = wid * _BPW

    @pl.loop(0, _NGROUPS)
    def _group(grp):
        p0 = grp * _G
        pltpu.sync_copy(idx_hbm.at[wid, pl.ds(p0, _G)], idx_v)
        descs = []
        for g in range(_G):
            for h in range(2):
                descs.append(
                    pltpu.async_copy(
                        tables_hbm.at[idx_v.at[g, h]], rows_v.at[g, h], gsem
                    )
                )
        for d in descs:
            d.wait()

        @pl.loop(0, _BPW, unroll=4)
        def _mul(b):
            for g in range(_G):
                out_v[b, g, :] = rows_v[g, 0, b, :] * rows_v[g, 1, b, :]

        pltpu.sync_copy(out_v, out_hbm.at[pl.ds(b0, _BPW), pl.ds(p0, _G)])


@jax.jit
def kernel(x, tables):
    x32 = x.astype(jnp.int32)
    idx_a = x32[:, _IU].T + jnp.asarray(_CONST_A)[:, None]
    idx_b = x32[:, _JU].T + jnp.asarray(_CONST_B)[:, None]
    idx = jnp.stack(
        [idx_a.reshape(_NPAIR, _NW, _BPW), idx_b.reshape(_NPAIR, _NW, _BPW)],
        axis=2,
    ).transpose(1, 0, 2, 3)
    tables_flat = tables.reshape(_F * _SUM, _D)

    mesh = plsc.VectorSubcoreMesh(core_axis_name="c", subcore_axis_name="s")
    f = functools.partial(
        pl.kernel,
        out_type=jax.ShapeDtypeStruct((_B, _NPAIR, _D), jnp.float32),
        mesh=mesh,
        scratch_types=[
            pltpu.VMEM((_G, 2, _BPW), jnp.int32),
            pltpu.VMEM((_G, 2, _BPW, _D), jnp.float32),
            pltpu.VMEM((_BPW, _G, _D), jnp.float32),
            pltpu.SemaphoreType.DMA,
        ],
        compiler_params=pltpu.CompilerParams(use_tc_tiling_on_sc=False),
    )(_ffm_body)
    return f(tables_flat, idx)
